# baseline (speedup 1.0000x reference)
import functools

import numpy as np
import jax
import jax.numpy as jnp
from jax import lax
from jax.experimental import pallas as pl
from jax.experimental.pallas import tpu as pltpu
from jax.experimental.pallas import tpu_sc as plsc

MAX_T = 1024
EMB_D = 256
NUM_FREQS = 10
OUT_D = EMB_D + 2 * NUM_FREQS
PAD_D = 280


def _table_body(emb_ref, out_ref):
    emb = emb_ref[:]
    t = lax.broadcasted_iota(jnp.int32, (MAX_T, NUM_FREQS), 0).astype(jnp.float32)
    j = lax.broadcasted_iota(jnp.int32, (MAX_T, NUM_FREQS), 1).astype(jnp.float32)
    raw = t * jnp.exp2(j) * np.float32(np.pi / MAX_T)
    pad = jnp.zeros((MAX_T, PAD_D - OUT_D), jnp.float32)
    out_ref[:] = jnp.concatenate([emb, jnp.sin(raw), jnp.cos(raw), pad], axis=1)


def _build_table(embed_table):
    return pl.pallas_call(
        _table_body,
        out_shape=jax.ShapeDtypeStruct((MAX_T, PAD_D), jnp.float32),
    )(embed_table)


def _sc_gather(idx, table):
    n = idx.shape[0]
    info = plsc.get_sparse_core_info()
    nc, ns, nl = info.num_cores, info.num_subcores, info.num_lanes
    nw = nc * ns
    b_per_w = n // nw
    chunk = 64
    n_chunks = b_per_w // chunk
    n_vec = OUT_D // nl
    mesh = plsc.VectorSubcoreMesh(core_axis_name="c", subcore_axis_name="s")

    @functools.partial(
        pl.kernel,
        out_type=jax.ShapeDtypeStruct((n * OUT_D,), jnp.float32),
        mesh=mesh,
        scratch_types=[
            pltpu.VMEM((b_per_w,), jnp.int32),
            pltpu.VMEM((2, chunk, PAD_D), jnp.float32),
            pltpu.VMEM((2, chunk * OUT_D), jnp.float32),
            pltpu.VMEM_SHARED((MAX_T, PAD_D), jnp.float32),
            pltpu.SemaphoreType.DMA,
            pltpu.SemaphoreType.DMA,
            pltpu.SemaphoreType.DMA,
            pltpu.SemaphoreType.DMA,
        ],
        compiler_params=pltpu.CompilerParams(use_tc_tiling_on_sc=False),
    )
    def gather_kernel(
        idx_hbm, table_hbm, out_hbm, idx_v, rows_v, flat_v, shared_v, g0, g1, w0, w1
    ):
        gsem = (g0, g1)
        wsem = (w0, w1)
        sid = lax.axis_index("s")
        wid = sid * nc + lax.axis_index("c")
        base = wid * b_per_w

        @pl.when(sid == 0)
        def _():
            pltpu.sync_copy(table_hbm, shared_v)

        plsc.subcore_barrier()
        pltpu.sync_copy(idx_hbm.at[pl.ds(base, b_per_w)], idx_v)

        def start_gather(k, b):
            pltpu.async_copy(
                shared_v.at[idx_v.at[pl.ds(k * chunk, chunk)]],
                rows_v.at[b],
                gsem[b],
            )

        def wait_gather(b):
            pltpu.make_async_copy(
                table_hbm.at[pl.ds(0, chunk)], rows_v.at[b], gsem[b]
            ).wait()

        def start_write(k, b):
            pltpu.async_copy(
                flat_v.at[b],
                out_hbm.at[pl.ds((base + k * chunk) * OUT_D, chunk * OUT_D)],
                wsem[b],
            )

        def wait_write(b):
            pltpu.make_async_copy(
                flat_v.at[b], out_hbm.at[pl.ds(0, chunk * OUT_D)], wsem[b]
            ).wait()

        start_gather(0, 0)

        def outer(i, carry):
            k0 = i * 2
            for b in range(2):
                k = k0 + b

                @pl.when(k + 1 < n_chunks)
                def _():
                    start_gather(k + 1, 1 - b)

                wait_gather(b)

                @pl.when(k >= 2)
                def _():
                    wait_write(b)

                def row_body(r, carry2):
                    dst = r * OUT_D
                    for v in range(n_vec):
                        flat_v[b, pl.ds(dst + v * nl, nl)] = rows_v[
                            b, r, pl.ds(v * nl, nl)
                        ]
                    flat_v[b, pl.ds(dst + OUT_D - nl, nl)] = rows_v[
                        b, r, pl.ds(OUT_D - nl, nl)
                    ]
                    return carry2

                lax.fori_loop(0, chunk, row_body, 0)
                start_write(k, b)
            return carry

        lax.fori_loop(0, n_chunks // 2, outer, 0)
        wait_write(0)
        wait_write(1)

    return gather_kernel(idx, table)


def kernel(delta_t, embed_table):
    batch = delta_t.shape[0]
    idx = delta_t.reshape(-1).astype(jnp.int32)
    fused = _build_table(embed_table)
    out_flat = _sc_gather(idx, fused)
    return out_flat.reshape(batch, -1)


# speedup vs baseline: 1.5238x; 1.5238x over previous
import functools

import numpy as np
import jax
import jax.numpy as jnp
from jax import lax
from jax.experimental import pallas as pl
from jax.experimental.pallas import tpu as pltpu
from jax.experimental.pallas import tpu_sc as plsc

MAX_T = 1024
EMB_D = 256
NUM_FREQS = 10
OUT_D = EMB_D + 2 * NUM_FREQS
PAD_D = 280


def _table_body(emb_ref, out_ref):
    emb = emb_ref[:]
    t = lax.broadcasted_iota(jnp.int32, (MAX_T, NUM_FREQS), 0).astype(jnp.float32)
    j = lax.broadcasted_iota(jnp.int32, (MAX_T, NUM_FREQS), 1).astype(jnp.float32)
    raw = t * jnp.exp2(j) * np.float32(np.pi / MAX_T)
    pad = jnp.zeros((MAX_T, PAD_D - OUT_D), jnp.float32)
    out_ref[:] = jnp.concatenate([emb, jnp.sin(raw), jnp.cos(raw), pad], axis=1)


def _build_table(embed_table):
    return pl.pallas_call(
        _table_body,
        out_shape=jax.ShapeDtypeStruct((MAX_T, PAD_D), jnp.float32),
    )(embed_table)


def _sc_gather(idx, table):
    n = idx.shape[0]
    info = plsc.get_sparse_core_info()
    nc, ns, nl = info.num_cores, info.num_subcores, info.num_lanes
    nw = nc * ns
    b_per_w = n // nw
    chunk = 64
    n_chunks = b_per_w // chunk
    n_vec = OUT_D // nl
    mesh = plsc.VectorSubcoreMesh(core_axis_name="c", subcore_axis_name="s")

    @functools.partial(
        pl.kernel,
        out_type=jax.ShapeDtypeStruct((n * OUT_D,), jnp.float32),
        mesh=mesh,
        scratch_types=[
            pltpu.VMEM((b_per_w,), jnp.int32),
            pltpu.VMEM((2, chunk, PAD_D), jnp.float32),
            pltpu.VMEM((2, chunk * OUT_D), jnp.float32),
            pltpu.VMEM_SHARED((MAX_T, PAD_D), jnp.float32),
            pltpu.SemaphoreType.DMA,
            pltpu.SemaphoreType.DMA,
            pltpu.SemaphoreType.DMA,
            pltpu.SemaphoreType.DMA,
        ],
        compiler_params=pltpu.CompilerParams(use_tc_tiling_on_sc=False),
    )
    def gather_kernel(
        idx_hbm, table_hbm, out_hbm, idx_v, rows_v, flat_v, shared_v, g0, g1, w0, w1
    ):
        gsem = (g0, g1)
        wsem = (w0, w1)
        sid = lax.axis_index("s")
        wid = sid * nc + lax.axis_index("c")
        base = wid * b_per_w

        @pl.when(sid == 0)
        def _():
            pltpu.sync_copy(table_hbm, shared_v)

        plsc.subcore_barrier()
        pltpu.sync_copy(idx_hbm.at[pl.ds(base, b_per_w)], idx_v)

        def start_gather(k, b):
            pltpu.async_copy(
                shared_v.at[idx_v.at[pl.ds(k * chunk, chunk)]],
                rows_v.at[b],
                gsem[b],
            )

        def wait_gather(b):
            pltpu.make_async_copy(
                table_hbm.at[pl.ds(0, chunk)], rows_v.at[b], gsem[b]
            ).wait()

        def start_write(k, b):
            pltpu.async_copy(
                flat_v.at[b],
                out_hbm.at[pl.ds((base + k * chunk) * OUT_D, chunk * OUT_D)],
                wsem[b],
            )

        def wait_write(b):
            pltpu.make_async_copy(
                flat_v.at[b], out_hbm.at[pl.ds(0, chunk * OUT_D)], wsem[b]
            ).wait()

        start_gather(0, 0)

        def outer(i, carry):
            k0 = i * 2
            for b in range(2):
                k = k0 + b

                @pl.when(k + 1 < n_chunks)
                def _():
                    start_gather(k + 1, 1 - b)

                wait_gather(b)

                @pl.when(k >= 2)
                def _():
                    wait_write(b)

                @plsc.parallel_loop(0, chunk, unroll=2)
                def row_body(r):
                    dst = r * OUT_D
                    for v in range(n_vec):
                        flat_v[b, pl.ds(dst + v * nl, nl)] = rows_v[
                            b, r, pl.ds(v * nl, nl)
                        ]
                    flat_v[b, pl.ds(dst + OUT_D - nl, nl)] = rows_v[
                        b, r, pl.ds(OUT_D - nl, nl)
                    ]
                start_write(k, b)
            return carry

        lax.fori_loop(0, n_chunks // 2, outer, 0)
        wait_write(0)
        wait_write(1)

    return gather_kernel(idx, table)


def kernel(delta_t, embed_table):
    batch = delta_t.shape[0]
    idx = delta_t.reshape(-1).astype(jnp.int32)
    fused = _build_table(embed_table)
    out_flat = _sc_gather(idx, fused)
    return out_flat.reshape(batch, -1)
